# baseline (speedup 1.0000x reference)
import jax
import jax.numpy as jnp
from jax import lax
from jax.experimental import pallas as pl
from jax.experimental.pallas import tpu as pltpu
from jax.experimental.pallas import tpu_sc as plsc

BATCH = 16384
HIST = 200
DIM = 32
NC, NS = 2, 16
NW = NC * NS
TBT = 4
CT = DIM // 8
H2 = HIST // 2


def _gather_body(idx_hbm, wl_hbm, o5_hbm,
                 idx_v0, idx_v1, rows0, rows1, rows2, rows3, tb0, tb1,
                 si0, si1, sg0, sg1, sg2, sg3, so0, so1):
    wid = lax.axis_index("s") * NC + lax.axis_index("c")
    bt0 = wid * TBT

    idx_v = (idx_v0, idx_v1)
    rows = (rows0, rows1, rows2, rows3)
    tb = (tb0, tb1)
    si = (si0, si1)
    sg = (sg0, sg1, sg2, sg3)
    so = (so0, so1)

    iota16 = lax.iota(jnp.int32, 16)
    rvecs = [iota16 + (k * 16) for k in range(8)]

    def idx_slice(h):
        return idx_hbm.at[h, pl.ds(bt0, TBT)]

    def fire_idx(h, islot):
        pltpu.async_copy(idx_slice(h), idx_v[islot], si[islot])

    def wait_idx(h, islot):
        pltpu.make_async_copy(idx_slice(h), idx_v[islot], si[islot]).wait()

    def fire_g(islot, uu, rs):
        pltpu.async_copy(wl_hbm.at[idx_v[islot].at[uu]], rows[rs], sg[rs])

    def wait_g(islot, uu, rs):
        pltpu.make_async_copy(wl_hbm.at[idx_v[islot].at[uu]], rows[rs], sg[rs]).wait()

    def fire_out(h, uu, s):
        for ct in range(CT):
            pltpu.async_copy(tb[s].at[ct], o5_hbm.at[h, ct, bt0 + uu], so[s])

    def wait_out(h, uu, s):
        for ct in range(CT):
            pltpu.make_async_copy(
                tb[s].at[ct], o5_hbm.at[h, ct, bt0 + uu], so[s]).wait()

    def transpose_unit(rs, ts):
        def cbody(c2, carry):
            for j in range(2):
                c = c2 * 2 + j
                csplat = jnp.full((16,), 0, jnp.int32) + c
                vals = [plsc.load_gather(rows[rs], [rvecs[k], csplat])
                        for k in range(8)]
                for k in range(8):
                    tb[ts][c // 8, c % 8, pl.ds(k * 16, 16)] = vals[k]
            return carry

        lax.fori_loop(0, DIM // 2, cbody, 0)

    fire_idx(0, 0)
    wait_idx(0, 0)
    for uu in range(TBT):
        fire_g(0, uu, uu)
    fire_idx(1, 1)

    def h2body(h2, carry):
        for hh in range(2):
            h = h2 * 2 + hh
            if hh == 0:
                wait_idx(h + 1, 1)
            else:
                @pl.when(h + 1 < HIST)
                def _():
                    wait_idx(h + 1, 0)

            for uu in range(TBT):
                s = uu % 2
                wait_g(hh, uu, uu)

                if hh == 0 and uu < 2:
                    @pl.when(h2 > 0)
                    def _():
                        wait_out(h, uu, s)
                else:
                    wait_out(h, uu, s)

                transpose_unit(uu, s)
                fire_out(h, uu, s)

                if hh == 0:
                    fire_g(1, uu, uu)
                else:
                    @pl.when(h + 1 < HIST)
                    def _():
                        fire_g(0, uu, uu)

            @pl.when(h + 2 < HIST)
            def _():
                fire_idx(h + 2, hh)
        return carry

    lax.fori_loop(0, H2, h2body, 0)

    wait_out(HIST - 1, TBT - 2, 0)
    wait_out(HIST - 1, TBT - 1, 1)


def kernel(input, weight):
    idxT3 = jnp.transpose(input).astype(jnp.int32).reshape(HIST, 128, 128)
    mesh = plsc.VectorSubcoreMesh(core_axis_name="c", subcore_axis_name="s")
    o5 = pl.kernel(
        _gather_body,
        mesh=mesh,
        out_type=jax.ShapeDtypeStruct((HIST, CT, 128, 8, 128), jnp.float32),
        scratch_types=[
            pltpu.VMEM((TBT, 128), jnp.int32),
            pltpu.VMEM((TBT, 128), jnp.int32),
            pltpu.VMEM((128, DIM), jnp.float32),
            pltpu.VMEM((128, DIM), jnp.float32),
            pltpu.VMEM((128, DIM), jnp.float32),
            pltpu.VMEM((128, DIM), jnp.float32),
            pltpu.VMEM((CT, 8, 128), jnp.float32),
            pltpu.VMEM((CT, 8, 128), jnp.float32),
            pltpu.SemaphoreType.DMA,
            pltpu.SemaphoreType.DMA,
            pltpu.SemaphoreType.DMA,
            pltpu.SemaphoreType.DMA,
            pltpu.SemaphoreType.DMA,
            pltpu.SemaphoreType.DMA,
            pltpu.SemaphoreType.DMA,
            pltpu.SemaphoreType.DMA,
        ],
        compiler_params=pltpu.CompilerParams(
            use_tc_tiling_on_sc=False, needs_layout_passes=False),
    )(idxT3, weight)
    return o5.transpose(2, 4, 0, 1, 3).reshape(BATCH, HIST, DIM)


# speedup vs baseline: 1.0739x; 1.0739x over previous
import jax
import jax.numpy as jnp
from jax import lax
from jax.experimental import pallas as pl
from jax.experimental.pallas import tpu as pltpu
from jax.experimental.pallas import tpu_sc as plsc

BATCH = 16384
HIST = 200
DIM = 32
TOTAL = BATCH * HIST
NC, NS = 2, 16
NW = NC * NS
NB = 4
CHUNK = NB * HIST
BAT_PER_W = BATCH // NW
CHUNKS_PER_W = BAT_PER_W // NB
SPLITS = ((0, 128), (128, 72))


def _gather_body(idx_hbm, table_hbm, out_hbm,
                 idx_v0, idx_v1, rows_v0, rows_v1,
                 si0, si1, sg0, sg1, so0, so1):
    wid = lax.axis_index("s") * NC + lax.axis_index("c")
    bat0 = wid * BAT_PER_W

    idx_v = (idx_v0, idx_v1)
    rows_v = (rows_v0, rows_v1)
    si = (si0, si1)
    sg = (sg0, sg1)
    so = (so0, so1)

    def idx_slice(g):
        return idx_hbm.at[pl.ds((bat0 + g * NB) * HIST, CHUNK)]

    def out_slice(g):
        return out_hbm.at[pl.ds(bat0 + g * NB, NB)]

    def fire_gathers(b):
        for row in range(NB):
            for off, ln in SPLITS:
                pltpu.async_copy(
                    table_hbm.at[idx_v[b].at[pl.ds(row * HIST + off, ln)]],
                    rows_v[b].at[row, pl.ds(off, ln)],
                    sg[b],
                )

    def drain_gathers(b):
        for row in range(NB):
            for off, ln in SPLITS:
                pltpu.make_async_copy(
                    table_hbm.at[idx_v[b].at[pl.ds(row * HIST + off, ln)]],
                    rows_v[b].at[row, pl.ds(off, ln)],
                    sg[b],
                ).wait()

    pltpu.async_copy(idx_slice(0), idx_v[0], si[0])

    def round_fn(r, carry):
        for b in range(2):
            g = r * 2 + b
            ob = 1 - b
            pltpu.make_async_copy(idx_slice(g), idx_v[b], si[b]).wait()

            @pl.when(r > 0)
            def _():
                pltpu.make_async_copy(rows_v[b], out_slice(g), so[b]).wait()

            fire_gathers(b)

            @pl.when(g >= 1)
            def _():
                drain_gathers(ob)
                pltpu.async_copy(rows_v[ob], out_slice(g - 1), so[ob])

            @pl.when(g + 1 < CHUNKS_PER_W)
            def _():
                pltpu.async_copy(idx_slice(g + 1), idx_v[ob], si[ob])
        return carry

    lax.fori_loop(0, CHUNKS_PER_W // 2, round_fn, 0)

    last = CHUNKS_PER_W - 1
    drain_gathers(1)
    pltpu.async_copy(rows_v[1], out_slice(last), so[1])
    pltpu.make_async_copy(rows_v[0], out_slice(last - 1), so[0]).wait()
    pltpu.make_async_copy(rows_v[1], out_slice(last), so[1]).wait()


def kernel(input, weight):
    idx1d = input.reshape(TOTAL).astype(jnp.int32)
    mesh = plsc.VectorSubcoreMesh(core_axis_name="c", subcore_axis_name="s")
    return pl.kernel(
        _gather_body,
        mesh=mesh,
        out_type=jax.ShapeDtypeStruct((BATCH, HIST, DIM), jnp.float32),
        scratch_types=[
            pltpu.VMEM((CHUNK,), jnp.int32),
            pltpu.VMEM((CHUNK,), jnp.int32),
            pltpu.VMEM((NB, HIST, DIM), jnp.float32),
            pltpu.VMEM((NB, HIST, DIM), jnp.float32),
            pltpu.SemaphoreType.DMA,
            pltpu.SemaphoreType.DMA,
            pltpu.SemaphoreType.DMA,
            pltpu.SemaphoreType.DMA,
            pltpu.SemaphoreType.DMA,
            pltpu.SemaphoreType.DMA,
        ],
        compiler_params=pltpu.CompilerParams(use_tc_tiling_on_sc=False),
    )(idx1d, weight)
